# Initial kernel scaffold; baseline (speedup 1.0000x reference)
import jax
import jax.numpy as jnp
from jax import lax
from jax.experimental import pallas as pl
from jax.experimental.pallas import tpu as pltpu
from jax.experimental.pallas import tpu_sc as plsc

_D = 128
_CB = 128
_NC = 2
_NS = 16
_NW = _NC * _NS


def _body(xT, t0, t1, t2, out, x_v, i0, i1, i2, g0, g1, g2, sem):
    E = xT.shape[1]
    nchunks = E // _CB
    wid = lax.axis_index("s") * _NC + lax.axis_index("c")
    nk = nchunks // _NW

    def chunk(k, _):
        c = wid + k * _NW
        base = c * _CB
        pltpu.sync_copy(xT.at[:, pl.ds(base, _CB)], x_v)
        for g in range(_CB // 16):
            sl = pl.ds(g * 16, 16)
            i0[sl] = (x_v[0, sl] * 2 + x_v[9, sl]) * 2 + x_v[10, sl]
            i1[sl] = ((x_v[1, sl] * 11 + x_v[2, sl]) * 11
                      + x_v[3, sl]) * 11 + x_v[4, sl]
            i2[sl] = ((x_v[5, sl] * 6 + x_v[6, sl]) * 6
                      + x_v[7, sl]) * 5 + x_v[8, sl]
        cp0 = pltpu.async_copy(t0.at[i0], g0, sem)
        cp1 = pltpu.async_copy(t1.at[i1], g1, sem)
        cp2 = pltpu.async_copy(t2.at[i2], g2, sem)
        cp0.wait()
        cp1.wait()
        cp2.wait()

        def acc(r, _):
            for j in range(_D // 16):
                sl = pl.ds(j * 16, 16)
                g0[r, sl] = g0[r, sl] + g1[r, sl] + g2[r, sl]
            return ()

        lax.fori_loop(0, _CB, acc, (), unroll=2)
        pltpu.sync_copy(g0, out.at[pl.ds(base, _CB), :])
        return ()

    lax.fori_loop(0, nk, chunk, ())


def kernel(x, W0, W1, W2, W3, W4, W5, W6, W7, W8, W9, W10):
    E = x.shape[0]
    t0 = (W0[:, None, None, :] + W9[None, :, None, :]
          + W10[None, None, :, :]).reshape(-1, _D)
    t1 = (W1[:, None, None, None, :] + W2[None, :, None, None, :]
          + W3[None, None, :, None, :]
          + W4[None, None, None, :, :]).reshape(-1, _D)
    t2 = (W5[:, None, None, None, :] + W6[None, :, None, None, :]
          + W7[None, None, :, None, :]
          + W8[None, None, None, :, :]).reshape(-1, _D)
    xT = x.T

    mesh = plsc.VectorSubcoreMesh(core_axis_name="c", subcore_axis_name="s")
    run = pl.kernel(
        _body,
        out_type=jax.ShapeDtypeStruct((E, _D), jnp.float32),
        mesh=mesh,
        scratch_types=[
            pltpu.VMEM((11, _CB), jnp.int32),
            pltpu.VMEM((_CB,), jnp.int32),
            pltpu.VMEM((_CB,), jnp.int32),
            pltpu.VMEM((_CB,), jnp.int32),
            pltpu.VMEM((_CB, _D), jnp.float32),
            pltpu.VMEM((_CB, _D), jnp.float32),
            pltpu.VMEM((_CB, _D), jnp.float32),
            pltpu.SemaphoreType.DMA,
        ],
    )
    return run(xT, t0, t1, t2)


# speedup vs baseline: 8.2031x; 8.2031x over previous
import jax
import jax.numpy as jnp
from jax import lax
from jax.experimental import pallas as pl
from jax.experimental.pallas import tpu as pltpu
from jax.experimental.pallas import tpu_sc as plsc

_D = 128
_CB = 128
_NC = 2
_NS = 16
_NW = _NC * _NS


def _body(xT, t0, t1, t2, out, x_v, i0, i1, i2, g0, g1, g2, sem):
    E = xT.shape[1]
    nchunks = E // _CB
    wid = lax.axis_index("s") * _NC + lax.axis_index("c")
    nk = (nchunks - wid + _NW - 1) // _NW

    def chunk(k, _):
        c = wid + k * _NW
        base = c * _CB
        pltpu.sync_copy(xT.at[:, pl.ds(base, _CB)], x_v)
        for g in range(_CB // 16):
            sl = pl.ds(g * 16, 16)
            i0[sl] = (x_v[0, sl] * 2 + x_v[9, sl]) * 2 + x_v[10, sl]
            i1[sl] = ((x_v[1, sl] * 11 + x_v[2, sl]) * 11
                      + x_v[3, sl]) * 11 + x_v[4, sl]
            i2[sl] = ((x_v[5, sl] * 6 + x_v[6, sl]) * 6
                      + x_v[7, sl]) * 5 + x_v[8, sl]
        cp0 = pltpu.async_copy(t0.at[i0], g0, sem)
        cp1 = pltpu.async_copy(t1.at[i1], g1, sem)
        cp2 = pltpu.async_copy(t2.at[i2], g2, sem)
        cp0.wait()
        cp1.wait()
        cp2.wait()

        def acc(r, _):
            for j in range(_D // 16):
                sl = pl.ds(j * 16, 16)
                g0[r, sl] = g0[r, sl] + g1[r, sl] + g2[r, sl]
            return ()

        lax.fori_loop(0, _CB, acc, (), unroll=2)
        pltpu.sync_copy(g0, out.at[pl.ds(base, _CB), :])
        return ()

    lax.fori_loop(0, nk, chunk, ())


def kernel(x, W0, W1, W2, W3, W4, W5, W6, W7, W8, W9, W10):
    E = x.shape[0]
    t0 = (W0[:, None, None, :] + W9[None, :, None, :]
          + W10[None, None, :, :]).reshape(-1, _D)
    t1 = (W1[:, None, None, None, :] + W2[None, :, None, None, :]
          + W3[None, None, :, None, :]
          + W4[None, None, None, :, :]).reshape(-1, _D)
    t2 = (W5[:, None, None, None, :] + W6[None, :, None, None, :]
          + W7[None, None, :, None, :]
          + W8[None, None, None, :, :]).reshape(-1, _D)
    xT = x.T

    mesh = plsc.VectorSubcoreMesh(core_axis_name="c", subcore_axis_name="s")
    run = pl.kernel(
        _body,
        out_type=jax.ShapeDtypeStruct((E, _D), jnp.float32),
        mesh=mesh,
        scratch_types=[
            pltpu.VMEM((11, _CB), jnp.int32),
            pltpu.VMEM((_CB,), jnp.int32),
            pltpu.VMEM((_CB,), jnp.int32),
            pltpu.VMEM((_CB,), jnp.int32),
            pltpu.VMEM((_CB, _D), jnp.float32),
            pltpu.VMEM((_CB, _D), jnp.float32),
            pltpu.VMEM((_CB, _D), jnp.float32),
            pltpu.SemaphoreType.DMA,
        ],
    )
    return run(xT, t0, t1, t2)


# speedup vs baseline: 14.1476x; 1.7247x over previous
import jax
import jax.numpy as jnp
from jax import lax
from jax.experimental import pallas as pl
from jax.experimental.pallas import tpu as pltpu
from jax.experimental.pallas import tpu_sc as plsc

_D = 128
_CB = 128
_NC = 2
_NS = 16
_NW = _NC * _NS


def _body(xT, t0, t1, t2, out, x_v, i0, i1, i2, g0, g1, g2, sem):
    E = xT.shape[1]
    nchunks = E // _CB
    wid = lax.axis_index("s") * _NC + lax.axis_index("c")
    nk = (nchunks - wid + _NW - 1) // _NW

    def chunk(k, _):
        c = wid + k * _NW
        base = c * _CB
        pltpu.sync_copy(xT.at[:, pl.ds(base, _CB)], x_v)
        for g in range(_CB // 16):
            sl = pl.ds(g * 16, 16)
            i0[sl] = (x_v[0, sl] * 2 + x_v[9, sl]) * 2 + x_v[10, sl]
            i1[sl] = ((x_v[1, sl] * 11 + x_v[2, sl]) * 11
                      + x_v[3, sl]) * 11 + x_v[4, sl]
            i2[sl] = ((x_v[5, sl] * 6 + x_v[6, sl]) * 6
                      + x_v[7, sl]) * 5 + x_v[8, sl]
        zeros = jnp.zeros((16,), jnp.float32)

        def zero(r, _):
            for j in range(_D // 16):
                g0[r, pl.ds(j * 16, 16)] = zeros
            return ()

        lax.fori_loop(0, _CB, zero, (), unroll=2)
        cp0 = pltpu.make_async_copy(t0.at[i0], g0, sem)
        cp1 = pltpu.make_async_copy(t1.at[i1], g0, sem)
        cp2 = pltpu.make_async_copy(t2.at[i2], g0, sem)
        cp0.start(add=True)
        cp1.start(add=True)
        cp2.start(add=True)
        cp0.wait()
        cp1.wait()
        cp2.wait()
        pltpu.sync_copy(g0, out.at[pl.ds(base, _CB), :])
        return ()

    lax.fori_loop(0, nk, chunk, ())


def kernel(x, W0, W1, W2, W3, W4, W5, W6, W7, W8, W9, W10):
    E = x.shape[0]
    t0 = (W0[:, None, None, :] + W9[None, :, None, :]
          + W10[None, None, :, :]).reshape(-1, _D)
    t1 = (W1[:, None, None, None, :] + W2[None, :, None, None, :]
          + W3[None, None, :, None, :]
          + W4[None, None, None, :, :]).reshape(-1, _D)
    t2 = (W5[:, None, None, None, :] + W6[None, :, None, None, :]
          + W7[None, None, :, None, :]
          + W8[None, None, None, :, :]).reshape(-1, _D)
    xT = x.T

    mesh = plsc.VectorSubcoreMesh(core_axis_name="c", subcore_axis_name="s")
    run = pl.kernel(
        _body,
        out_type=jax.ShapeDtypeStruct((E, _D), jnp.float32),
        mesh=mesh,
        scratch_types=[
            pltpu.VMEM((11, _CB), jnp.int32),
            pltpu.VMEM((_CB,), jnp.int32),
            pltpu.VMEM((_CB,), jnp.int32),
            pltpu.VMEM((_CB,), jnp.int32),
            pltpu.VMEM((_CB, _D), jnp.float32),
            pltpu.VMEM((_CB, _D), jnp.float32),
            pltpu.VMEM((_CB, _D), jnp.float32),
            pltpu.SemaphoreType.DMA,
        ],
    )
    return run(xT, t0, t1, t2)


# speedup vs baseline: 14.3431x; 1.0138x over previous
import jax
import jax.numpy as jnp
from jax import lax
from jax.experimental import pallas as pl
from jax.experimental.pallas import tpu as pltpu
from jax.experimental.pallas import tpu_sc as plsc

_D = 128
_CB = 128
_NC = 2
_NS = 16
_NW = _NC * _NS


def _compute_idx(xv, i3):
    i0, i1, i2 = i3
    for g in range(_CB // 16):
        sl = pl.ds(g * 16, 16)
        i0[sl] = (xv[0, sl] * 2 + xv[9, sl]) * 2 + xv[10, sl]
        i1[sl] = ((xv[1, sl] * 11 + xv[2, sl]) * 11
                  + xv[3, sl]) * 11 + xv[4, sl]
        i2[sl] = ((xv[5, sl] * 6 + xv[6, sl]) * 6
                  + xv[7, sl]) * 5 + xv[8, sl]


def _zero_g(gb):
    zeros = jnp.zeros((16,), jnp.float32)

    def zero(r, _):
        for j in range(_D // 16):
            gb[r, pl.ds(j * 16, 16)] = zeros
        return ()

    lax.fori_loop(0, _CB, zero, (), unroll=4)


def _body(xT, t0, t1, t2, out,
          xv0, xv1, i00, i01, i02, i10, i11, i12, g0, g1,
          xs0, xs1, gs0, gs1, os0, os1):
    E = xT.shape[1]
    nchunks = E // _CB
    wid = lax.axis_index("s") * _NC + lax.axis_index("c")
    nk = (nchunks - wid + _NW - 1) // _NW
    xvs = (xv0, xv1)
    idx = ((i00, i01, i02), (i10, i11, i12))
    gbufs = (g0, g1)
    xsem = (xs0, xs1)
    gsem = (gs0, gs1)
    osem = (os0, os1)
    tables = (t0, t1, t2)

    def ebase(k):
        return (wid + k * _NW) * _CB

    def x_copy(k, b):
        return pltpu.make_async_copy(
            xT.at[:, pl.ds(ebase(k), _CB)], xvs[b], xsem[b])

    def gathers(b, do_start):
        cps = [pltpu.make_async_copy(tables[t].at[idx[b][t]], gbufs[b],
                                     gsem[b]) for t in range(3)]
        for cp in cps:
            if do_start:
                cp.start(add=True)
            else:
                cp.wait()

    def out_copy(k, b, do_start):
        cp = pltpu.make_async_copy(
            gbufs[b], out.at[pl.ds(ebase(k), _CB), :], osem[b])
        if do_start:
            cp.start()
        else:
            cp.wait()

    x_copy(0, 0).start()
    x_copy(0, 0).wait()
    _compute_idx(xvs[0], idx[0])
    _zero_g(gbufs[0])
    gathers(0, True)
    x_copy(1, 1).start()

    def step(k, b):
        bn = 1 - b

        @pl.when(k + 1 < nk)
        def _():
            x_copy(k + 1, bn).wait()
            _compute_idx(xvs[bn], idx[bn])

        @pl.when(k + 2 < nk)
        def _():
            x_copy(k + 2, b).start()

        gathers(b, False)
        out_copy(k, b, True)

        @pl.when(k >= 1)
        def _():
            out_copy(k - 1, bn, False)

        @pl.when(k + 1 < nk)
        def _():
            _zero_g(gbufs[bn])
            gathers(bn, True)

    def pair(p, _):
        k0 = 2 * p

        @pl.when(k0 < nk)
        def _():
            step(k0, 0)

        @pl.when(k0 + 1 < nk)
        def _():
            step(k0 + 1, 1)

        return ()

    lax.fori_loop(0, (nk + 1) // 2, pair, ())

    @pl.when((nk - 1) % 2 == 0)
    def _():
        out_copy(nk - 1, 0, False)

    @pl.when((nk - 1) % 2 == 1)
    def _():
        out_copy(nk - 1, 1, False)


def kernel(x, W0, W1, W2, W3, W4, W5, W6, W7, W8, W9, W10):
    E = x.shape[0]
    t0 = (W0[:, None, None, :] + W9[None, :, None, :]
          + W10[None, None, :, :]).reshape(-1, _D)
    t1 = (W1[:, None, None, None, :] + W2[None, :, None, None, :]
          + W3[None, None, :, None, :]
          + W4[None, None, None, :, :]).reshape(-1, _D)
    t2 = (W5[:, None, None, None, :] + W6[None, :, None, None, :]
          + W7[None, None, :, None, :]
          + W8[None, None, None, :, :]).reshape(-1, _D)
    xT = x.T

    mesh = plsc.VectorSubcoreMesh(core_axis_name="c", subcore_axis_name="s")
    run = pl.kernel(
        _body,
        out_type=jax.ShapeDtypeStruct((E, _D), jnp.float32),
        mesh=mesh,
        scratch_types=[
            pltpu.VMEM((11, _CB), jnp.int32),
            pltpu.VMEM((11, _CB), jnp.int32),
            pltpu.VMEM((_CB,), jnp.int32),
            pltpu.VMEM((_CB,), jnp.int32),
            pltpu.VMEM((_CB,), jnp.int32),
            pltpu.VMEM((_CB,), jnp.int32),
            pltpu.VMEM((_CB,), jnp.int32),
            pltpu.VMEM((_CB,), jnp.int32),
            pltpu.VMEM((_CB, _D), jnp.float32),
            pltpu.VMEM((_CB, _D), jnp.float32),
            pltpu.SemaphoreType.DMA,
            pltpu.SemaphoreType.DMA,
            pltpu.SemaphoreType.DMA,
            pltpu.SemaphoreType.DMA,
            pltpu.SemaphoreType.DMA,
            pltpu.SemaphoreType.DMA,
        ],
    )
    return run(xT, t0, t1, t2)


# speedup vs baseline: 20.7970x; 1.4500x over previous
import jax
import jax.numpy as jnp
from jax import lax
from jax.experimental import pallas as pl
from jax.experimental.pallas import tpu as pltpu
from jax.experimental.pallas import tpu_sc as plsc

_D = 128
_CB = 128
_R0 = 256
_R2 = 16
_NC = 2
_NS = 16
_NW = _NC * _NS


def _compute_idx(xv, i3, kb):
    i0, i1, i2 = i3
    iota = lax.iota(jnp.int32, 16)
    rep = kb + iota
    for g in range(_CB // 16):
        sl = pl.ds(g * 16, 16)
        rg = rep + g * 16
        i0[sl] = ((xv[0, sl] * 2 + xv[9, sl]) * 2 + xv[10, sl]
                  + (rg & (_R0 - 1)) * 176)
        i1[sl] = ((xv[1, sl] * 11 + xv[2, sl]) * 11
                  + xv[3, sl]) * 11 + xv[4, sl]
        i2[sl] = (((xv[5, sl] * 6 + xv[6, sl]) * 6
                   + xv[7, sl]) * 5 + xv[8, sl]
                  + (rg & (_R2 - 1)) * 1980)


def _zero_g(gb):
    zeros = jnp.zeros((16,), jnp.float32)

    def zero(r, _):
        for j in range(_D // 16):
            gb[r, pl.ds(j * 16, 16)] = zeros
        return ()

    lax.fori_loop(0, _CB, zero, (), unroll=4)


def _body(xT, t0, t1, t2, out,
          xv0, xv1, i00, i01, i02, i10, i11, i12, g0, g1,
          xs0, xs1, gs0, gs1, os0, os1):
    E = xT.shape[1]
    nchunks = E // _CB
    wid = lax.axis_index("s") * _NC + lax.axis_index("c")
    nk = (nchunks - wid + _NW - 1) // _NW
    xvs = (xv0, xv1)
    idx = ((i00, i01, i02), (i10, i11, i12))
    gbufs = (g0, g1)
    xsem = (xs0, xs1)
    gsem = (gs0, gs1)
    osem = (os0, os1)
    tables = (t0, t1, t2)

    def ebase(k):
        return (wid + k * _NW) * _CB

    def x_copy(k, b):
        return pltpu.make_async_copy(
            xT.at[:, pl.ds(ebase(k), _CB)], xvs[b], xsem[b])

    def gathers(b, do_start):
        cps = [pltpu.make_async_copy(tables[t].at[idx[b][t]], gbufs[b],
                                     gsem[b]) for t in range(3)]
        for cp in cps:
            if do_start:
                cp.start(add=True)
            else:
                cp.wait()

    def out_copy(k, b, do_start):
        cp = pltpu.make_async_copy(
            gbufs[b], out.at[pl.ds(ebase(k), _CB), :], osem[b])
        if do_start:
            cp.start()
        else:
            cp.wait()

    x_copy(0, 0).start()
    x_copy(0, 0).wait()
    _compute_idx(xvs[0], idx[0], ebase(0))
    _zero_g(gbufs[0])
    gathers(0, True)
    x_copy(1, 1).start()

    def step(k, b):
        bn = 1 - b

        @pl.when(k + 1 < nk)
        def _():
            x_copy(k + 1, bn).wait()
            _compute_idx(xvs[bn], idx[bn], ebase(k + 1))

        @pl.when(k + 2 < nk)
        def _():
            x_copy(k + 2, b).start()

        gathers(b, False)
        out_copy(k, b, True)

        @pl.when(k >= 1)
        def _():
            out_copy(k - 1, bn, False)

        @pl.when(k + 1 < nk)
        def _():
            _zero_g(gbufs[bn])
            gathers(bn, True)

    def pair(p, _):
        k0 = 2 * p

        @pl.when(k0 < nk)
        def _():
            step(k0, 0)

        @pl.when(k0 + 1 < nk)
        def _():
            step(k0 + 1, 1)

        return ()

    lax.fori_loop(0, (nk + 1) // 2, pair, ())

    @pl.when((nk - 1) % 2 == 0)
    def _():
        out_copy(nk - 1, 0, False)

    @pl.when((nk - 1) % 2 == 1)
    def _():
        out_copy(nk - 1, 1, False)


def kernel(x, W0, W1, W2, W3, W4, W5, W6, W7, W8, W9, W10):
    E = x.shape[0]
    t0 = (W0[:, None, None, :] + W9[None, :, None, :]
          + W10[None, None, :, :]).reshape(-1, _D)
    t1 = (W1[:, None, None, None, :] + W2[None, :, None, None, :]
          + W3[None, None, :, None, :]
          + W4[None, None, None, :, :]).reshape(-1, _D)
    t2 = (W5[:, None, None, None, :] + W6[None, :, None, None, :]
          + W7[None, None, :, None, :]
          + W8[None, None, None, :, :]).reshape(-1, _D)
    t0 = jnp.tile(t0, (_R0, 1))
    t2 = jnp.tile(t2, (_R2, 1))
    xT = x.T

    mesh = plsc.VectorSubcoreMesh(core_axis_name="c", subcore_axis_name="s")
    run = pl.kernel(
        _body,
        out_type=jax.ShapeDtypeStruct((E, _D), jnp.float32),
        mesh=mesh,
        scratch_types=[
            pltpu.VMEM((11, _CB), jnp.int32),
            pltpu.VMEM((11, _CB), jnp.int32),
            pltpu.VMEM((_CB,), jnp.int32),
            pltpu.VMEM((_CB,), jnp.int32),
            pltpu.VMEM((_CB,), jnp.int32),
            pltpu.VMEM((_CB,), jnp.int32),
            pltpu.VMEM((_CB,), jnp.int32),
            pltpu.VMEM((_CB,), jnp.int32),
            pltpu.VMEM((_CB, _D), jnp.float32),
            pltpu.VMEM((_CB, _D), jnp.float32),
            pltpu.SemaphoreType.DMA,
            pltpu.SemaphoreType.DMA,
            pltpu.SemaphoreType.DMA,
            pltpu.SemaphoreType.DMA,
            pltpu.SemaphoreType.DMA,
            pltpu.SemaphoreType.DMA,
        ],
    )
    return run(xT, t0, t1, t2)


# speedup vs baseline: 24.3143x; 1.1691x over previous
import jax
import jax.numpy as jnp
from jax import lax
from jax.experimental import pallas as pl
from jax.experimental.pallas import tpu as pltpu
from jax.experimental.pallas import tpu_sc as plsc

_D = 128
_CB = 128
_R0 = 256
_R2 = 16
_NC = 2
_NS = 16
_NW = _NC * _NS


def _compute_idx(xv, i3, kb):
    i0, i1, i2 = i3
    iota = lax.iota(jnp.int32, 16)
    rep = kb + iota
    for g in range(_CB // 16):
        sl = pl.ds(g * 16, 16)
        rg = rep + g * 16
        i0[sl] = ((xv[0, sl] * 2 + xv[9, sl]) * 2 + xv[10, sl]
                  + (rg & (_R0 - 1)) * 176)
        i1[sl] = ((xv[1, sl] * 11 + xv[2, sl]) * 11
                  + xv[3, sl]) * 11 + xv[4, sl]
        i2[sl] = (((xv[5, sl] * 6 + xv[6, sl]) * 6
                   + xv[7, sl]) * 5 + xv[8, sl]
                  + (rg & (_R2 - 1)) * 1980)


def _zero_g(gb):
    zeros = jnp.zeros((16,), jnp.float32)

    def zero(r, _):
        for j in range(_D // 16):
            gb[r, pl.ds(j * 16, 16)] = zeros
        return ()

    lax.fori_loop(0, _CB, zero, (), unroll=4)


def _body(xT, t0, t1, t2, out,
          xv0, xv1, i00, i01, i02, i10, i11, i12, g0, g1,
          xs0, xs1, gs0, gs1, os0, os1):
    E = xT.shape[1]
    nchunks = E // _CB
    wid = lax.axis_index("s") * _NC + lax.axis_index("c")
    nk = (nchunks - wid + _NW - 1) // _NW
    xvs = (xv0, xv1)
    idx = ((i00, i01, i02), (i10, i11, i12))
    gbufs = (g0, g1)
    xsem = (xs0, xs1)
    gsem = (gs0, gs1)
    osem = (os0, os1)
    tables = (t0, t1, t2)

    def ebase(k):
        return (wid + k * _NW) * _CB

    def x_copy(k, b):
        return pltpu.make_async_copy(
            xT.at[:, pl.ds(ebase(k), _CB)], xvs[b], xsem[b])

    def gathers(b, do_start):
        cps = [pltpu.make_async_copy(tables[t].at[idx[b][t]], gbufs[b],
                                     gsem[b]) for t in range(3)]
        for cp in cps:
            if do_start:
                cp.start(add=True)
            else:
                cp.wait()

    def out_copy(k, b, do_start):
        cp = pltpu.make_async_copy(
            gbufs[b], out.at[pl.ds(ebase(k), _CB), :], osem[b])
        if do_start:
            cp.start()
        else:
            cp.wait()

    x_copy(0, 0).start()
    x_copy(0, 0).wait()
    _compute_idx(xvs[0], idx[0], ebase(0))
    _zero_g(gbufs[0])
    gathers(0, True)
    x_copy(1, 1).start()

    def step(k, b):
        bn = 1 - b

        @pl.when(k >= 1)
        def _():
            out_copy(k - 1, bn, False)

        @pl.when(k + 1 < nk)
        def _():
            x_copy(k + 1, bn).wait()
            _compute_idx(xvs[bn], idx[bn], ebase(k + 1))
            _zero_g(gbufs[bn])
            gathers(bn, True)

        @pl.when(k + 2 < nk)
        def _():
            x_copy(k + 2, b).start()

        gathers(b, False)
        out_copy(k, b, True)

    def pair(p, _):
        k0 = 2 * p

        @pl.when(k0 < nk)
        def _():
            step(k0, 0)

        @pl.when(k0 + 1 < nk)
        def _():
            step(k0 + 1, 1)

        return ()

    lax.fori_loop(0, (nk + 1) // 2, pair, ())

    @pl.when((nk - 1) % 2 == 0)
    def _():
        out_copy(nk - 1, 0, False)

    @pl.when((nk - 1) % 2 == 1)
    def _():
        out_copy(nk - 1, 1, False)


def kernel(x, W0, W1, W2, W3, W4, W5, W6, W7, W8, W9, W10):
    E = x.shape[0]
    t0 = (W0[:, None, None, :] + W9[None, :, None, :]
          + W10[None, None, :, :]).reshape(-1, _D)
    t1 = (W1[:, None, None, None, :] + W2[None, :, None, None, :]
          + W3[None, None, :, None, :]
          + W4[None, None, None, :, :]).reshape(-1, _D)
    t2 = (W5[:, None, None, None, :] + W6[None, :, None, None, :]
          + W7[None, None, :, None, :]
          + W8[None, None, None, :, :]).reshape(-1, _D)
    t0 = jnp.tile(t0, (_R0, 1))
    t2 = jnp.tile(t2, (_R2, 1))
    xT = x.T

    mesh = plsc.VectorSubcoreMesh(core_axis_name="c", subcore_axis_name="s")
    run = pl.kernel(
        _body,
        out_type=jax.ShapeDtypeStruct((E, _D), jnp.float32),
        mesh=mesh,
        scratch_types=[
            pltpu.VMEM((11, _CB), jnp.int32),
            pltpu.VMEM((11, _CB), jnp.int32),
            pltpu.VMEM((_CB,), jnp.int32),
            pltpu.VMEM((_CB,), jnp.int32),
            pltpu.VMEM((_CB,), jnp.int32),
            pltpu.VMEM((_CB,), jnp.int32),
            pltpu.VMEM((_CB,), jnp.int32),
            pltpu.VMEM((_CB,), jnp.int32),
            pltpu.VMEM((_CB, _D), jnp.float32),
            pltpu.VMEM((_CB, _D), jnp.float32),
            pltpu.SemaphoreType.DMA,
            pltpu.SemaphoreType.DMA,
            pltpu.SemaphoreType.DMA,
            pltpu.SemaphoreType.DMA,
            pltpu.SemaphoreType.DMA,
            pltpu.SemaphoreType.DMA,
        ],
    )
    return run(xT, t0, t1, t2)
